# baseline (speedup 1.0000x reference)
import functools

import jax
import jax.numpy as jnp
from jax import lax
from jax.experimental import pallas as pl
from jax.experimental.pallas import tpu as pltpu
from jax.experimental.pallas import tpu_sc as plsc

NC = 2
NS = 16
L = 16
NW = NC * NS

CHUNK = 1024
NB = 6
DEPTH = 3


@functools.cache
def _sc_edge_kernel(n_nodes: int, n_acc: int, e_pad: int, chunks_per_tile: int):
    edges_per_tile = e_pad // NW
    zslice = n_acc // NS

    mesh = plsc.VectorSubcoreMesh(core_axis_name="c", subcore_axis_name="s")

    idx_buf = pltpu.VMEM((CHUNK,), jnp.int32)
    val_buf = pltpu.VMEM((CHUNK,), jnp.float32)

    @functools.partial(
        pl.kernel,
        mesh=mesh,
        compiler_params=pltpu.CompilerParams(needs_layout_passes=False),
        out_type=jax.ShapeDtypeStruct((NC, n_acc), jnp.float32),
        scratch_types=[
            pltpu.VMEM((n_nodes,), jnp.float32),
            [idx_buf] * NB,
            [idx_buf] * NB,
            [val_buf] * NB,
            [val_buf] * NB,
            pltpu.VMEM_SHARED((n_acc,), jnp.float32),
            [pltpu.SemaphoreType.DMA] * NB,
            [pltpu.SemaphoreType.DMA] * NB,
        ],
    )
    def sc_kernel(nodes_hbm, row_hbm, col_hbm, ev_hbm, zeros_hbm, out_hbm,
                  nodes_v, rbufs, cbufs, ebufs, lbufs, acc_sh,
                  in_sems, sc_sems):
        c = lax.axis_index("c")
        s = lax.axis_index("s")
        wid = s * NC + c
        base_edge = wid * edges_per_tile

        def in_descs(ci, slot):
            sl = pl.ds(base_edge + ci * CHUNK, CHUNK)
            sem = in_sems[slot]
            return (
                pltpu.make_async_copy(row_hbm.at[sl], rbufs[slot], sem),
                pltpu.make_async_copy(col_hbm.at[sl], cbufs[slot], sem),
                pltpu.make_async_copy(ev_hbm.at[sl], ebufs[slot], sem),
            )

        def sc_desc(slot):
            return pltpu.make_async_copy(
                lbufs[slot], acc_sh.at[cbufs[slot]], sc_sems[slot])

        pltpu.sync_copy(nodes_hbm, nodes_v)
        pltpu.sync_copy(zeros_hbm.at[pl.ds(s * zslice, zslice)],
                        acc_sh.at[pl.ds(s * zslice, zslice)])
        plsc.subcore_barrier()

        for ci in range(DEPTH):
            for d in in_descs(ci, ci % NB):
                d.start()

        @pl.loop(0, chunks_per_tile, step=NB)
        def _group(bi):
            for p in range(NB):
                ci = bi + p
                fslot = (p + DEPTH) % NB
                if p >= NB - DEPTH:
                    sc_desc(fslot).wait()

                    @pl.when(ci + DEPTH < chunks_per_tile)
                    def _():
                        for d in in_descs(ci + DEPTH, fslot):
                            d.start()
                else:
                    @pl.when(bi > 0)
                    def _():
                        sc_desc(fslot).wait()

                    for d in in_descs(ci + DEPTH, fslot):
                        d.start()

                for d in in_descs(ci, p):
                    d.wait()
                @plsc.parallel_loop(0, CHUNK, L, unroll=4)
                def _k(k):
                    sl = pl.ds(k, L)
                    ir = rbufs[p][sl]
                    ic = cbufs[p][sl]
                    e = ebufs[p][sl]
                    src = plsc.load_gather(nodes_v, [ir])
                    dst = plsc.load_gather(nodes_v, [ic])
                    m = (src * e) > 0
                    lbufs[p][sl] = jnp.where(m, (dst - src) / e,
                                             jnp.float32(0.0))

                sc_desc(p).start(add=True)

        for q in range(NB - DEPTH, NB):
            sc_desc(q % NB).wait()

        plsc.subcore_barrier()

        @pl.when(s == 0)
        def _():
            pltpu.sync_copy(acc_sh, out_hbm.at[c])

    return sc_kernel


@functools.cache
def _tc_sum_kernel(n_acc: int):
    def body(p_ref, o_ref):
        o_ref[...] = p_ref[0] + p_ref[1]

    return pl.pallas_call(
        body,
        out_shape=jax.ShapeDtypeStruct((n_acc // 128, 128), jnp.float32),
    )


def kernel(x, edge_index, edge_attr):
    n = x.shape[0]
    e_cnt = edge_index.shape[1]

    nodes = x[:, 0]
    row = edge_index[0].astype(jnp.int32)
    col = edge_index[1].astype(jnp.int32)
    ev = edge_attr[:, 0]

    grain = NW * CHUNK * NB
    e_pad = -(-e_cnt // grain) * grain
    pad = e_pad - e_cnt
    if pad:
        row = jnp.concatenate([row, jnp.zeros((pad,), jnp.int32)])
        col = jnp.concatenate([col, jnp.zeros((pad,), jnp.int32)])
        ev = jnp.concatenate([ev, jnp.zeros((pad,), jnp.float32)])

    n_acc = -(-n // 2048) * 2048
    sc = _sc_edge_kernel(n, n_acc, e_pad, e_pad // (NW * CHUNK))
    partial = sc(nodes, row, col, ev, jnp.zeros((n_acc,), jnp.float32))

    summed = _tc_sum_kernel(n_acc)(partial.reshape(NC, n_acc // 128, 128))
    return summed.reshape(-1)[:n]


# speedup vs baseline: 1.9465x; 1.9465x over previous
import functools

import jax
import jax.numpy as jnp
from jax import lax
from jax.experimental import pallas as pl
from jax.experimental.pallas import tpu as pltpu
from jax.experimental.pallas import tpu_sc as plsc

NC = 2
NS = 16
L = 16
NW = NC * NS

CHUNK = 1024
NB = 5
DEPTH = 2


@functools.cache
def _sc_edge_kernel(n_nodes: int, n_acc: int, edges_per_tile: int):
    zslice = n_acc // NS
    full_chunks = edges_per_tile // CHUNK
    loop_chunks = full_chunks - full_chunks % NB
    post = full_chunks - loop_chunks
    tail = edges_per_tile - full_chunks * CHUNK

    mesh = plsc.VectorSubcoreMesh(core_axis_name="c", subcore_axis_name="s")

    idx_buf = pltpu.VMEM((CHUNK,), jnp.int32)
    val_buf = pltpu.VMEM((CHUNK,), jnp.float32)
    tail_i = pltpu.VMEM((max(tail, L),), jnp.int32)
    tail_v = pltpu.VMEM((max(tail, L),), jnp.float32)

    @functools.partial(
        pl.kernel,
        mesh=mesh,
        compiler_params=pltpu.CompilerParams(needs_layout_passes=False),
        out_type=jax.ShapeDtypeStruct((NC, n_acc), jnp.float32),
        scratch_types=[
            pltpu.VMEM((n_nodes,), jnp.float32),
            [idx_buf] * NB,
            [idx_buf] * NB,
            [val_buf] * NB,
            [val_buf] * NB,
            [tail_i, tail_i, tail_v, tail_v],
            pltpu.VMEM_SHARED((n_acc,), jnp.float32),
            [pltpu.SemaphoreType.DMA] * NB,
            [pltpu.SemaphoreType.DMA] * NB,
            pltpu.SemaphoreType.DMA,
        ],
    )
    def sc_kernel(nodes_hbm, row_hbm, col_hbm, ev_hbm, zeros_hbm, out_hbm,
                  nodes_v, rbufs, cbufs, ebufs, lbufs, tbufs, acc_sh,
                  in_sems, sc_sems, t_sem):
        c = lax.axis_index("c")
        s = lax.axis_index("s")
        wid = s * NC + c
        base_edge = wid * edges_per_tile

        def in_descs(ci, slot):
            sl = pl.ds(base_edge + ci * CHUNK, CHUNK)
            sem = in_sems[slot]
            return (
                pltpu.make_async_copy(row_hbm.at[sl], rbufs[slot], sem),
                pltpu.make_async_copy(col_hbm.at[sl], cbufs[slot], sem),
                pltpu.make_async_copy(ev_hbm.at[sl], ebufs[slot], sem),
            )

        def tail_descs():
            sl = pl.ds(base_edge + full_chunks * CHUNK, tail)
            return (
                pltpu.make_async_copy(row_hbm.at[sl], tbufs[0], t_sem),
                pltpu.make_async_copy(col_hbm.at[sl], tbufs[1], t_sem),
                pltpu.make_async_copy(ev_hbm.at[sl], tbufs[2], t_sem),
            )

        def sc_desc(slot):
            return pltpu.make_async_copy(
                lbufs[slot], acc_sh.at[cbufs[slot]], sc_sems[slot])

        def edge_block(rbuf, cbuf, ebuf, lbuf, n_edges):
            @plsc.parallel_loop(0, n_edges, L, unroll=4)
            def _k(k):
                sl = pl.ds(k, L)
                ir = rbuf[sl]
                ic = cbuf[sl]
                e = ebuf[sl]
                src = plsc.load_gather(nodes_v, [ir])
                dst = plsc.load_gather(nodes_v, [ic])
                m = (src * e) > 0
                lbuf[sl] = jnp.where(m, (dst - src) / e, jnp.float32(0.0))

        pltpu.sync_copy(nodes_hbm, nodes_v)
        pltpu.sync_copy(zeros_hbm.at[pl.ds(s * zslice, zslice)],
                        acc_sh.at[pl.ds(s * zslice, zslice)])
        plsc.subcore_barrier()

        if tail:
            for d in tail_descs():
                d.start()

        for ci in range(min(DEPTH, full_chunks)):
            for d in in_descs(ci, ci % NB):
                d.start()

        @pl.loop(0, loop_chunks, step=NB)
        def _group(bi):
            for p in range(NB):
                ci = bi + p
                fslot = (p + DEPTH) % NB
                if p >= NB - DEPTH:
                    sc_desc(fslot).wait()

                    @pl.when(ci + DEPTH < full_chunks)
                    def _():
                        for d in in_descs(ci + DEPTH, fslot):
                            d.start()
                else:
                    @pl.when(bi > 0)
                    def _():
                        sc_desc(fslot).wait()

                    for d in in_descs(ci + DEPTH, fslot):
                        d.start()

                for d in in_descs(ci, p):
                    d.wait()
                edge_block(rbufs[p], cbufs[p], ebufs[p], lbufs[p], CHUNK)
                sc_desc(p).start(add=True)

        for i in range(post):
            ci = loop_chunks + i
            slot = ci % NB
            if i >= DEPTH:
                sc_desc(slot).wait()
                for d in in_descs(ci, slot):
                    d.start()
            for d in in_descs(ci, slot):
                d.wait()
            edge_block(rbufs[slot], cbufs[slot], ebufs[slot], lbufs[slot],
                       CHUNK)
            sc_desc(slot).start(add=True)

        if tail:
            for d in tail_descs():
                d.wait()
            edge_block(tbufs[0], tbufs[1], tbufs[2], tbufs[3], tail)
            pltpu.sync_copy(tbufs[3], acc_sh.at[tbufs[1]], add=True)

        drained = set(range(max(0, loop_chunks - (NB - DEPTH))))
        drained |= {loop_chunks + i - NB for i in range(DEPTH, post)}
        for cj in range(full_chunks):
            if cj not in drained:
                sc_desc(cj % NB).wait()

        plsc.subcore_barrier()

        pltpu.sync_copy(acc_sh.at[pl.ds(s * zslice, zslice)],
                        out_hbm.at[c, pl.ds(s * zslice, zslice)])

    return sc_kernel


@functools.cache
def _tc_sum_kernel(n_acc: int):
    def body(p_ref, o_ref):
        o_ref[...] = p_ref[0] + p_ref[1]

    return pl.pallas_call(
        body,
        out_shape=jax.ShapeDtypeStruct((n_acc // 128, 128), jnp.float32),
    )


def kernel(x, edge_index, edge_attr):
    n = x.shape[0]
    e_cnt = edge_index.shape[1]

    nodes = x[:, 0]
    row = edge_index[0].astype(jnp.int32)
    col = edge_index[1].astype(jnp.int32)
    ev = edge_attr[:, 0]

    grain = NW * L
    e_pad = -(-e_cnt // grain) * grain
    pad = e_pad - e_cnt
    if pad:
        row = jnp.concatenate([row, jnp.zeros((pad,), jnp.int32)])
        col = jnp.concatenate([col, jnp.zeros((pad,), jnp.int32)])
        ev = jnp.concatenate([ev, jnp.zeros((pad,), jnp.float32)])

    n_acc = -(-n // 2048) * 2048
    sc = _sc_edge_kernel(n, n_acc, e_pad // NW)
    partial = sc(nodes, row, col, ev, jnp.zeros((n_acc,), jnp.float32))

    summed = _tc_sum_kernel(n_acc)(partial.reshape(NC, n_acc // 128, 128))
    return summed.reshape(-1)[:n]


# speedup vs baseline: 1.9688x; 1.0115x over previous
import functools

import jax
import jax.numpy as jnp
from jax import lax
from jax.experimental import pallas as pl
from jax.experimental.pallas import tpu as pltpu
from jax.experimental.pallas import tpu_sc as plsc

NC = 2
NS = 16
L = 16
NW = NC * NS

CHUNK = 1024
NB = 5
DEPTH = 2


@functools.cache
def _sc_edge_kernel(n_nodes: int, n_acc: int, edges_per_tile: int):
    zslice = n_acc // NS
    full_chunks = edges_per_tile // CHUNK
    loop_chunks = full_chunks - full_chunks % NB
    post = full_chunks - loop_chunks
    tail = edges_per_tile - full_chunks * CHUNK

    mesh = plsc.VectorSubcoreMesh(core_axis_name="c", subcore_axis_name="s")

    idx_buf = pltpu.VMEM((CHUNK,), jnp.int32)
    val_buf = pltpu.VMEM((CHUNK,), jnp.float32)
    tail_i = pltpu.VMEM((max(tail, L),), jnp.int32)
    tail_v = pltpu.VMEM((max(tail, L),), jnp.float32)

    @functools.partial(
        pl.kernel,
        mesh=mesh,
        compiler_params=pltpu.CompilerParams(needs_layout_passes=False),
        out_type=jax.ShapeDtypeStruct((NC, n_acc), jnp.float32),
        scratch_types=[
            pltpu.VMEM((n_nodes,), jnp.float32),
            [idx_buf] * NB,
            [idx_buf] * NB,
            [val_buf] * NB,
            [val_buf] * NB,
            [tail_i, tail_i, tail_v, tail_v],
            pltpu.VMEM_SHARED((n_acc,), jnp.float32),
            [pltpu.SemaphoreType.DMA] * NB,
            [pltpu.SemaphoreType.DMA] * NB,
            pltpu.SemaphoreType.DMA,
            pltpu.SemaphoreType.DMA,
        ],
    )
    def sc_kernel(nodes_hbm, row_hbm, col_hbm, ev_hbm, zeros_hbm, out_hbm,
                  nodes_v, rbufs, cbufs, ebufs, lbufs, tbufs, acc_sh,
                  in_sems, sc_sems, t_sem, n_sem):
        c = lax.axis_index("c")
        s = lax.axis_index("s")
        wid = s * NC + c
        base_edge = wid * edges_per_tile

        def in_descs(ci, slot):
            sl = pl.ds(base_edge + ci * CHUNK, CHUNK)
            sem = in_sems[slot]
            return (
                pltpu.make_async_copy(row_hbm.at[sl], rbufs[slot], sem),
                pltpu.make_async_copy(col_hbm.at[sl], cbufs[slot], sem),
                pltpu.make_async_copy(ev_hbm.at[sl], ebufs[slot], sem),
            )

        def tail_descs():
            sl = pl.ds(base_edge + full_chunks * CHUNK, tail)
            return (
                pltpu.make_async_copy(row_hbm.at[sl], tbufs[0], t_sem),
                pltpu.make_async_copy(col_hbm.at[sl], tbufs[1], t_sem),
                pltpu.make_async_copy(ev_hbm.at[sl], tbufs[2], t_sem),
            )

        def sc_desc(slot):
            return pltpu.make_async_copy(
                lbufs[slot], acc_sh.at[cbufs[slot]], sc_sems[slot])

        def edge_block(rbuf, cbuf, ebuf, lbuf, n_edges):
            @plsc.parallel_loop(0, n_edges, L, unroll=4)
            def _k(k):
                sl = pl.ds(k, L)
                ir = rbuf[sl]
                ic = cbuf[sl]
                e = ebuf[sl]
                src = plsc.load_gather(nodes_v, [ir])
                dst = plsc.load_gather(nodes_v, [ic])
                m = (src * e) > 0
                lbuf[sl] = jnp.where(m, (dst - src) / e, jnp.float32(0.0))

        nsplit = next(k for k in (10, 8, 5, 4, 2, 1)
                      if n_nodes % k == 0 and (n_nodes // k) % 8 == 0)
        npart = n_nodes // nsplit
        node_descs = [
            pltpu.make_async_copy(nodes_hbm.at[pl.ds(i * npart, npart)],
                                  nodes_v.at[pl.ds(i * npart, npart)],
                                  n_sem)
            for i in range(nsplit)
        ]
        for d in node_descs:
            d.start()

        if tail:
            for d in tail_descs():
                d.start()

        for ci in range(min(DEPTH, full_chunks)):
            for d in in_descs(ci, ci % NB):
                d.start()

        pltpu.sync_copy(zeros_hbm.at[pl.ds(s * zslice, zslice)],
                        acc_sh.at[pl.ds(s * zslice, zslice)])
        for d in node_descs:
            d.wait()
        plsc.subcore_barrier()

        @pl.loop(0, loop_chunks, step=NB)
        def _group(bi):
            for p in range(NB):
                ci = bi + p
                fslot = (p + DEPTH) % NB
                if p >= NB - DEPTH:
                    sc_desc(fslot).wait()

                    @pl.when(ci + DEPTH < full_chunks)
                    def _():
                        for d in in_descs(ci + DEPTH, fslot):
                            d.start()
                else:
                    @pl.when(bi > 0)
                    def _():
                        sc_desc(fslot).wait()

                    for d in in_descs(ci + DEPTH, fslot):
                        d.start()

                for d in in_descs(ci, p):
                    d.wait()
                edge_block(rbufs[p], cbufs[p], ebufs[p], lbufs[p], CHUNK)
                sc_desc(p).start(add=True)

        for i in range(post):
            ci = loop_chunks + i
            slot = ci % NB
            if i >= DEPTH:
                sc_desc(slot).wait()
                for d in in_descs(ci, slot):
                    d.start()
            for d in in_descs(ci, slot):
                d.wait()
            edge_block(rbufs[slot], cbufs[slot], ebufs[slot], lbufs[slot],
                       CHUNK)
            sc_desc(slot).start(add=True)

        if tail:
            for d in tail_descs():
                d.wait()
            edge_block(tbufs[0], tbufs[1], tbufs[2], tbufs[3], tail)
            pltpu.sync_copy(tbufs[3], acc_sh.at[tbufs[1]], add=True)

        drained = set(range(max(0, loop_chunks - (NB - DEPTH))))
        drained |= {loop_chunks + i - NB for i in range(DEPTH, post)}
        for cj in range(full_chunks):
            if cj not in drained:
                sc_desc(cj % NB).wait()

        plsc.subcore_barrier()

        pltpu.sync_copy(acc_sh.at[pl.ds(s * zslice, zslice)],
                        out_hbm.at[c, pl.ds(s * zslice, zslice)])

    return sc_kernel


@functools.cache
def _tc_sum_kernel(n_acc: int):
    def body(p_ref, o_ref):
        o_ref[...] = p_ref[0] + p_ref[1]

    return pl.pallas_call(
        body,
        out_shape=jax.ShapeDtypeStruct((n_acc // 128, 128), jnp.float32),
    )


def kernel(x, edge_index, edge_attr):
    n = x.shape[0]
    e_cnt = edge_index.shape[1]

    nodes = x[:, 0]
    row = edge_index[0].astype(jnp.int32)
    col = edge_index[1].astype(jnp.int32)
    ev = edge_attr[:, 0]

    grain = NW * L
    e_pad = -(-e_cnt // grain) * grain
    pad = e_pad - e_cnt
    if pad:
        row = jnp.concatenate([row, jnp.zeros((pad,), jnp.int32)])
        col = jnp.concatenate([col, jnp.zeros((pad,), jnp.int32)])
        ev = jnp.concatenate([ev, jnp.zeros((pad,), jnp.float32)])

    n_acc = -(-n // 2048) * 2048
    sc = _sc_edge_kernel(n, n_acc, e_pad // NW)
    partial = sc(nodes, row, col, ev, jnp.zeros((n_acc,), jnp.float32))

    summed = _tc_sum_kernel(n_acc)(partial.reshape(NC, n_acc // 128, 128))
    return summed.reshape(-1)[:n]
